# baseline (speedup 1.0000x reference)
import jax
import jax.numpy as jnp
from jax.experimental import pallas as pl
from jax.experimental.pallas import tpu as pltpu

S, D, NF, E, K, H = 2048, 1024, 8, 8, 2, 1024
NS = 3
EPS = 1e-05
NC = 2
TT = 256
SPC = S // NC
TPC = SPC // TT


def _moe_body(hid_ref, feat_ref, lng_ref, lnb_ref, wr_ref, br_ref,
              w1_ref, b1_ref, w2_ref, b2_ref, out_ref,
              h_scr, hn_scr, hnl_scr, g_scr, acc_scr,
              w1b_scr, w1l_scr, w2b_scr, w2l_scr):
    s = pl.program_id(1)
    e = pl.program_id(2)
    i = pl.program_id(3)
    rows = pl.ds(i * TT, TT)

    @pl.when(i == 0)
    def _cast_weights():
        w1f = w1_ref[0, 0]
        w1h = w1f.astype(jnp.bfloat16)
        w1b_scr[...] = w1h
        w1l_scr[...] = (w1f - w1h.astype(jnp.float32)).astype(jnp.bfloat16)
        w2f = w2_ref[0, 0]
        w2h = w2f.astype(jnp.bfloat16)
        w2b_scr[...] = w2h
        w2l_scr[...] = (w2f - w2h.astype(jnp.float32)).astype(jnp.bfloat16)

    @pl.when(e == 0)
    def _router():
        @pl.when(s == 0)
        def _():
            h_scr[rows, :] = hid_ref[rows, :]

        @pl.when(s > 0)
        def _():
            h_scr[rows, :] = h_scr[rows, :] + acc_scr[rows, :]

        h = h_scr[rows, :]
        mu = jnp.mean(h, axis=1, keepdims=True)
        var = jnp.mean((h - mu) ** 2, axis=1, keepdims=True)
        hn = (h - mu) / jnp.sqrt(var + EPS) * lng_ref[0, 0, :] + lnb_ref[0, 0, :]
        hnh = hn.astype(jnp.bfloat16)
        hn_scr[rows, :] = hnh
        hnl_scr[rows, :] = (hn - hnh.astype(jnp.float32)).astype(jnp.bfloat16)

        logits = (jnp.dot(hn, wr_ref[0, :D, :],
                          preferred_element_type=jnp.float32)
                  + jnp.dot(feat_ref[rows, :], wr_ref[0, D:, :],
                            preferred_element_type=jnp.float32)
                  + br_ref[0, 0, :])
        m = jnp.max(logits, axis=1, keepdims=True)
        ex = jnp.exp(logits - m)
        probs = ex / jnp.sum(ex, axis=1, keepdims=True)

        idx = jax.lax.broadcasted_iota(jnp.int32, (TT, E), 1)
        m1 = jnp.max(probs, axis=1, keepdims=True)
        i1 = jnp.min(jnp.where(probs == m1, idx, E), axis=1, keepdims=True)
        oh1 = idx == i1
        probs2 = jnp.where(oh1, -1.0, probs)
        m2 = jnp.max(probs2, axis=1, keepdims=True)
        i2 = jnp.min(jnp.where(probs2 == m2, idx, E), axis=1, keepdims=True)
        oh2 = idx == i2
        denom = m1 + m2 + 1e-9
        g_scr[rows, :] = (jnp.where(oh1, m1, 0.0) + jnp.where(oh2, m2, 0.0)) / denom

    idx = jax.lax.broadcasted_iota(jnp.int32, (TT, E), 1)
    ge = jnp.sum(jnp.where(idx == e, g_scr[rows, :], 0.0),
                 axis=1, keepdims=True)

    def _accumulate(contrib):
        @pl.when(e == 0)
        def _():
            acc_scr[rows, :] = contrib

        @pl.when(e > 0)
        def _():
            acc_scr[rows, :] = acc_scr[rows, :] + contrib

    hnb = hn_scr[rows, :]

    @pl.when(s < NS - 1)
    def _precise():
        hnl = hnl_scr[rows, :]
        a1 = (jnp.dot(hnb, w1b_scr[...], preferred_element_type=jnp.float32)
              + jnp.dot(hnb, w1l_scr[...], preferred_element_type=jnp.float32)
              + jnp.dot(hnl, w1b_scr[...], preferred_element_type=jnp.float32)
              + b1_ref[0, 0, :])
        eh = jax.nn.gelu(a1)
        ehh = eh.astype(jnp.bfloat16)
        ehl = (eh - ehh.astype(jnp.float32)).astype(jnp.bfloat16)
        eo = (jnp.dot(ehh, w2b_scr[...], preferred_element_type=jnp.float32)
              + jnp.dot(ehh, w2l_scr[...], preferred_element_type=jnp.float32)
              + jnp.dot(ehl, w2b_scr[...], preferred_element_type=jnp.float32))
        _accumulate(ge * (eo + b2_ref[0, 0, :]))

    @pl.when(s == NS - 1)
    def _fast():
        a1 = jnp.dot(hnb, w1b_scr[...],
                     preferred_element_type=jnp.float32) + b1_ref[0, 0, :]
        eh = jax.nn.gelu(a1)
        eo = jnp.dot(eh.astype(jnp.bfloat16), w2b_scr[...],
                     preferred_element_type=jnp.float32)
        _accumulate(ge * (eo + b2_ref[0, 0, :]))

    @pl.when((s == NS - 1) & (e == E - 1))
    def _():
        out_ref[rows, :] = h_scr[rows, :] + acc_scr[rows, :]


def _run_moe(hid, feat, lng, lnb, wr, br, w1, b1, w2, b2):
    grid = (NC, NS, E, TPC)
    return pl.pallas_call(
        _moe_body,
        grid=grid,
        in_specs=[
            pl.BlockSpec((SPC, D), lambda c, s, e, i: (c, 0)),
            pl.BlockSpec((SPC, NF), lambda c, s, e, i: (c, 0)),
            pl.BlockSpec((1, 1, D), lambda c, s, e, i: (s, 0, 0)),
            pl.BlockSpec((1, 1, D), lambda c, s, e, i: (s, 0, 0)),
            pl.BlockSpec((1, D + NF, E), lambda c, s, e, i: (s, 0, 0)),
            pl.BlockSpec((1, 1, E), lambda c, s, e, i: (s, 0, 0)),
            pl.BlockSpec((1, 1, D, H), lambda c, s, e, i: (s, e, 0, 0)),
            pl.BlockSpec((1, 1, H), lambda c, s, e, i: (s * E + e, 0, 0)),
            pl.BlockSpec((1, 1, H, D), lambda c, s, e, i: (s, e, 0, 0)),
            pl.BlockSpec((1, 1, D), lambda c, s, e, i: (s * E + e, 0, 0)),
        ],
        out_specs=pl.BlockSpec((SPC, D), lambda c, s, e, i: (c, 0)),
        out_shape=jax.ShapeDtypeStruct((S, D), jnp.float32),
        scratch_shapes=[
            pltpu.VMEM((SPC, D), jnp.float32),
            pltpu.VMEM((SPC, D), jnp.bfloat16),
            pltpu.VMEM((SPC, D), jnp.bfloat16),
            pltpu.VMEM((SPC, E), jnp.float32),
            pltpu.VMEM((SPC, D), jnp.float32),
            pltpu.VMEM((D, H), jnp.bfloat16),
            pltpu.VMEM((D, H), jnp.bfloat16),
            pltpu.VMEM((H, D), jnp.bfloat16),
            pltpu.VMEM((H, D), jnp.bfloat16),
        ],
        compiler_params=pltpu.CompilerParams(
            dimension_semantics=("parallel", "arbitrary", "arbitrary",
                                 "arbitrary"),
        ),
    )(hid, feat, lng, lnb, wr, br, w1, b1, w2, b2)


def kernel(hidden, feat,
           ln_g0, ln_b0, Wr0, br0, W1_0, b1_0, W2_0, b2_0,
           ln_g1, ln_b1, Wr1, br1, W1_1, b1_1, W2_1, b2_1,
           ln_g2, ln_b2, Wr2, br2, W1_2, b1_2, W2_2, b2_2):
    lng = jnp.stack([ln_g0, ln_g1, ln_g2]).reshape(NS, 1, D)
    lnb = jnp.stack([ln_b0, ln_b1, ln_b2]).reshape(NS, 1, D)
    wr = jnp.stack([Wr0, Wr1, Wr2])
    br = jnp.stack([br0, br1, br2]).reshape(NS, 1, E)
    w1 = jnp.stack([W1_0, W1_1, W1_2])
    b1 = jnp.stack([b1_0, b1_1, b1_2]).reshape(NS * E, 1, H)
    w2 = jnp.stack([W2_0, W2_1, W2_2])
    b2 = jnp.stack([b2_0, b2_1, b2_2]).reshape(NS * E, 1, D)
    out = _run_moe(hidden.reshape(S, D), feat.reshape(S, NF),
                   lng, lnb, wr, br, w1, b1, w2, b2)
    return out.reshape(hidden.shape)


# speedup vs baseline: 1.4275x; 1.4275x over previous
import jax
import jax.numpy as jnp
from jax.experimental import pallas as pl
from jax.experimental.pallas import tpu as pltpu

S, D, NF, E, K, H = 2048, 1024, 8, 8, 2, 1024
NS = 3
EPS = 1e-05
NC = 2
TT = 256
SPC = S // NC
TPC = SPC // TT


def _moe_body(hid_ref, feat_ref, lng_ref, lnb_ref, wr_ref, br_ref,
              w1_ref, b1_ref, w2_ref, b2_ref, out_ref,
              h_scr, hn_scr, g_scr, acc_scr, w1b_scr, w2b_scr):
    s = pl.program_id(1)
    e = pl.program_id(2)
    i = pl.program_id(3)
    rows = pl.ds(i * TT, TT)

    @pl.when(i == 0)
    def _cast_weights():
        w1b_scr[...] = w1_ref[0, 0].astype(jnp.bfloat16)
        w2b_scr[...] = w2_ref[0, 0].astype(jnp.bfloat16)

    @pl.when(e == 0)
    def _router():
        @pl.when(s == 0)
        def _():
            h_scr[rows, :] = hid_ref[rows, :]

        @pl.when(s > 0)
        def _():
            h_scr[rows, :] = h_scr[rows, :] + acc_scr[rows, :]

        h = h_scr[rows, :]
        mu = jnp.mean(h, axis=1, keepdims=True)
        var = jnp.mean((h - mu) ** 2, axis=1, keepdims=True)
        hn = (h - mu) / jnp.sqrt(var + EPS) * lng_ref[0, 0, :] + lnb_ref[0, 0, :]
        hnh = hn.astype(jnp.bfloat16)
        hn_scr[rows, :] = hnh

        logits = (jnp.dot(hnh, wr_ref[0, :D, :].astype(jnp.bfloat16),
                          preferred_element_type=jnp.float32)
                  + jnp.dot(feat_ref[rows, :].astype(jnp.bfloat16),
                            wr_ref[0, D:, :].astype(jnp.bfloat16),
                            preferred_element_type=jnp.float32)
                  + br_ref[0, 0, :])
        m = jnp.max(logits, axis=1, keepdims=True)
        ex = jnp.exp(logits - m)
        probs = ex / jnp.sum(ex, axis=1, keepdims=True)

        idx = jax.lax.broadcasted_iota(jnp.int32, (TT, E), 1)
        m1 = jnp.max(probs, axis=1, keepdims=True)
        i1 = jnp.min(jnp.where(probs == m1, idx, E), axis=1, keepdims=True)
        oh1 = idx == i1
        probs2 = jnp.where(oh1, -1.0, probs)
        m2 = jnp.max(probs2, axis=1, keepdims=True)
        i2 = jnp.min(jnp.where(probs2 == m2, idx, E), axis=1, keepdims=True)
        oh2 = idx == i2
        denom = m1 + m2 + 1e-9
        g_scr[rows, :] = (jnp.where(oh1, m1, 0.0) + jnp.where(oh2, m2, 0.0)) / denom

    idx = jax.lax.broadcasted_iota(jnp.int32, (TT, E), 1)
    ge = jnp.sum(jnp.where(idx == e, g_scr[rows, :], 0.0),
                 axis=1, keepdims=True)

    def _accumulate(contrib):
        @pl.when(e == 0)
        def _():
            acc_scr[rows, :] = contrib

        @pl.when(e > 0)
        def _():
            acc_scr[rows, :] = acc_scr[rows, :] + contrib

    hnb = hn_scr[rows, :]
    a1 = jnp.dot(hnb, w1b_scr[...],
                 preferred_element_type=jnp.float32) + b1_ref[0, 0, :]
    eh = jax.nn.gelu(a1)
    eo = jnp.dot(eh.astype(jnp.bfloat16), w2b_scr[...],
                 preferred_element_type=jnp.float32)
    _accumulate(ge * (eo + b2_ref[0, 0, :]))

    @pl.when((s == NS - 1) & (e == E - 1))
    def _():
        out_ref[rows, :] = h_scr[rows, :] + acc_scr[rows, :]


def _run_moe(hid, feat, lng, lnb, wr, br, w1, b1, w2, b2):
    grid = (NC, NS, E, TPC)
    return pl.pallas_call(
        _moe_body,
        grid=grid,
        in_specs=[
            pl.BlockSpec((SPC, D), lambda c, s, e, i: (c, 0)),
            pl.BlockSpec((SPC, NF), lambda c, s, e, i: (c, 0)),
            pl.BlockSpec((1, 1, D), lambda c, s, e, i: (s, 0, 0)),
            pl.BlockSpec((1, 1, D), lambda c, s, e, i: (s, 0, 0)),
            pl.BlockSpec((1, D + NF, E), lambda c, s, e, i: (s, 0, 0)),
            pl.BlockSpec((1, 1, E), lambda c, s, e, i: (s, 0, 0)),
            pl.BlockSpec((1, 1, D, H), lambda c, s, e, i: (s, e, 0, 0)),
            pl.BlockSpec((1, 1, H), lambda c, s, e, i: (s * E + e, 0, 0)),
            pl.BlockSpec((1, 1, H, D), lambda c, s, e, i: (s, e, 0, 0)),
            pl.BlockSpec((1, 1, D), lambda c, s, e, i: (s * E + e, 0, 0)),
        ],
        out_specs=pl.BlockSpec((SPC, D), lambda c, s, e, i: (c, 0)),
        out_shape=jax.ShapeDtypeStruct((S, D), jnp.float32),
        scratch_shapes=[
            pltpu.VMEM((SPC, D), jnp.float32),
            pltpu.VMEM((SPC, D), jnp.bfloat16),
            pltpu.VMEM((SPC, E), jnp.float32),
            pltpu.VMEM((SPC, D), jnp.float32),
            pltpu.VMEM((D, H), jnp.bfloat16),
            pltpu.VMEM((H, D), jnp.bfloat16),
        ],
        compiler_params=pltpu.CompilerParams(
            dimension_semantics=("parallel", "arbitrary", "arbitrary",
                                 "arbitrary"),
        ),
    )(hid, feat, lng, lnb, wr, br, w1, b1, w2, b2)


def kernel(hidden, feat,
           ln_g0, ln_b0, Wr0, br0, W1_0, b1_0, W2_0, b2_0,
           ln_g1, ln_b1, Wr1, br1, W1_1, b1_1, W2_1, b2_1,
           ln_g2, ln_b2, Wr2, br2, W1_2, b1_2, W2_2, b2_2):
    lng = jnp.stack([ln_g0, ln_g1, ln_g2]).reshape(NS, 1, D)
    lnb = jnp.stack([ln_b0, ln_b1, ln_b2]).reshape(NS, 1, D)
    wr = jnp.stack([Wr0, Wr1, Wr2])
    br = jnp.stack([br0, br1, br2]).reshape(NS, 1, E)
    w1 = jnp.stack([W1_0, W1_1, W1_2])
    b1 = jnp.stack([b1_0, b1_1, b1_2]).reshape(NS * E, 1, H)
    w2 = jnp.stack([W2_0, W2_1, W2_2])
    b2 = jnp.stack([b2_0, b2_1, b2_2]).reshape(NS * E, 1, D)
    out = _run_moe(hidden.reshape(S, D), feat.reshape(S, NF),
                   lng, lnb, wr, br, w1, b1, w2, b2)
    return out.reshape(hidden.shape)


# speedup vs baseline: 1.5450x; 1.0823x over previous
import jax
import jax.numpy as jnp
from jax.experimental import pallas as pl
from jax.experimental.pallas import tpu as pltpu

S, D, NF, E, K, H = 2048, 1024, 8, 8, 2, 1024
NS = 3
EPS = 1e-05
NC = 2
TT = 256
SPC = S // NC
TPC = SPC // TT


def _moe_body(hid_ref, feat_ref, lng_ref, lnb_ref, wr_ref, br_ref,
              w1_ref, b1_ref, w2_ref, b2_ref, out_ref,
              h_scr, hn_scr, g_scr, acc_scr):
    s = pl.program_id(1)
    e = pl.program_id(2)
    i = pl.program_id(3)
    rows = pl.ds(i * TT, TT)

    @pl.when(e == 0)
    def _router():
        @pl.when(s == 0)
        def _():
            h_scr[rows, :] = hid_ref[rows, :]

        @pl.when(s > 0)
        def _():
            h_scr[rows, :] = h_scr[rows, :] + acc_scr[rows, :]

        h = h_scr[rows, :]
        mu = jnp.mean(h, axis=1, keepdims=True)
        var = jnp.mean((h - mu) ** 2, axis=1, keepdims=True)
        hn = (h - mu) / jnp.sqrt(var + EPS) * lng_ref[0, 0, :] + lnb_ref[0, 0, :]
        hnh = hn.astype(jnp.bfloat16)
        hn_scr[rows, :] = hnh

        logits = (jnp.dot(hnh, wr_ref[0, :D, :].astype(jnp.bfloat16),
                          preferred_element_type=jnp.float32)
                  + jnp.dot(feat_ref[rows, :].astype(jnp.bfloat16),
                            wr_ref[0, D:, :].astype(jnp.bfloat16),
                            preferred_element_type=jnp.float32)
                  + br_ref[0, 0, :])
        m = jnp.max(logits, axis=1, keepdims=True)
        ex = jnp.exp(logits - m)
        probs = ex / jnp.sum(ex, axis=1, keepdims=True)

        idx = jax.lax.broadcasted_iota(jnp.int32, (TT, E), 1)
        m1 = jnp.max(probs, axis=1, keepdims=True)
        i1 = jnp.min(jnp.where(probs == m1, idx, E), axis=1, keepdims=True)
        oh1 = idx == i1
        probs2 = jnp.where(oh1, -1.0, probs)
        m2 = jnp.max(probs2, axis=1, keepdims=True)
        i2 = jnp.min(jnp.where(probs2 == m2, idx, E), axis=1, keepdims=True)
        oh2 = idx == i2
        denom = m1 + m2 + 1e-9
        g_scr[rows, :] = (jnp.where(oh1, m1, 0.0) + jnp.where(oh2, m2, 0.0)) / denom

    idx = jax.lax.broadcasted_iota(jnp.int32, (TT, E), 1)
    ge = jnp.sum(jnp.where(idx == e, g_scr[rows, :], 0.0),
                 axis=1, keepdims=True)

    def _accumulate(contrib):
        @pl.when(e == 0)
        def _():
            acc_scr[rows, :] = contrib

        @pl.when(e > 0)
        def _():
            acc_scr[rows, :] = acc_scr[rows, :] + contrib

    hnb = hn_scr[rows, :]
    a1 = jnp.dot(hnb, w1_ref[0, 0],
                 preferred_element_type=jnp.float32) + b1_ref[0, 0, :]
    eh = jax.nn.gelu(a1)
    eo = jnp.dot(eh.astype(jnp.bfloat16), w2_ref[0, 0],
                 preferred_element_type=jnp.float32)
    _accumulate(ge * (eo + b2_ref[0, 0, :]))

    @pl.when((s == NS - 1) & (e == E - 1))
    def _():
        out_ref[rows, :] = h_scr[rows, :] + acc_scr[rows, :]


def _run_moe(hid, feat, lng, lnb, wr, br, w1, b1, w2, b2):
    grid = (NC, NS, E, TPC)
    return pl.pallas_call(
        _moe_body,
        grid=grid,
        in_specs=[
            pl.BlockSpec((SPC, D), lambda c, s, e, i: (c, 0)),
            pl.BlockSpec((SPC, NF), lambda c, s, e, i: (c, 0)),
            pl.BlockSpec((1, 1, D), lambda c, s, e, i: (s, 0, 0)),
            pl.BlockSpec((1, 1, D), lambda c, s, e, i: (s, 0, 0)),
            pl.BlockSpec((1, D + NF, E), lambda c, s, e, i: (s, 0, 0)),
            pl.BlockSpec((1, 1, E), lambda c, s, e, i: (s, 0, 0)),
            pl.BlockSpec((1, 1, D, H), lambda c, s, e, i: (s, e, 0, 0)),
            pl.BlockSpec((1, 1, H), lambda c, s, e, i: (s * E + e, 0, 0)),
            pl.BlockSpec((1, 1, H, D), lambda c, s, e, i: (s, e, 0, 0)),
            pl.BlockSpec((1, 1, D), lambda c, s, e, i: (s * E + e, 0, 0)),
        ],
        out_specs=pl.BlockSpec((SPC, D), lambda c, s, e, i: (c, 0)),
        out_shape=jax.ShapeDtypeStruct((S, D), jnp.float32),
        scratch_shapes=[
            pltpu.VMEM((SPC, D), jnp.float32),
            pltpu.VMEM((SPC, D), jnp.bfloat16),
            pltpu.VMEM((SPC, E), jnp.float32),
            pltpu.VMEM((SPC, D), jnp.float32),
        ],
        compiler_params=pltpu.CompilerParams(
            dimension_semantics=("parallel", "arbitrary", "arbitrary",
                                 "arbitrary"),
        ),
    )(hid, feat, lng, lnb, wr, br, w1, b1, w2, b2)


def kernel(hidden, feat,
           ln_g0, ln_b0, Wr0, br0, W1_0, b1_0, W2_0, b2_0,
           ln_g1, ln_b1, Wr1, br1, W1_1, b1_1, W2_1, b2_1,
           ln_g2, ln_b2, Wr2, br2, W1_2, b1_2, W2_2, b2_2):
    lng = jnp.stack([ln_g0, ln_g1, ln_g2]).reshape(NS, 1, D)
    lnb = jnp.stack([ln_b0, ln_b1, ln_b2]).reshape(NS, 1, D)
    wr = jnp.stack([Wr0, Wr1, Wr2])
    br = jnp.stack([br0, br1, br2]).reshape(NS, 1, E)
    w1 = jnp.stack([W1_0, W1_1, W1_2]).astype(jnp.bfloat16)
    b1 = jnp.stack([b1_0, b1_1, b1_2]).reshape(NS * E, 1, H)
    w2 = jnp.stack([W2_0, W2_1, W2_2]).astype(jnp.bfloat16)
    b2 = jnp.stack([b2_0, b2_1, b2_2]).reshape(NS * E, 1, D)
    out = _run_moe(hidden.reshape(S, D), feat.reshape(S, NF),
                   lng, lnb, wr, br, w1, b1, w2, b2)
    return out.reshape(hidden.shape)
